# Initial kernel scaffold; baseline (speedup 1.0000x reference)
import jax
import jax.numpy as jnp
from jax.experimental import pallas as pl


def kernel(x, edge_index, W1, b1, W2, b2, Wp, bp):
    raise NotImplementedError("write your pallas kernel here")


# speedup vs baseline: 11.0696x; 11.0696x over previous
import functools

import jax
import jax.numpy as jnp
from jax import lax
from jax.experimental import pallas as pl
from jax.experimental.pallas import tpu as pltpu
from jax.experimental.pallas import tpu_sc as plsc

NUM_CORES = 2
NUM_SUBCORES = 16
NUM_WORKERS = NUM_CORES * NUM_SUBCORES

G = 100
CH = 20


def _mp_sc_kernel(n_nodes, n_edges, h, with_deg):
    assert n_edges % (NUM_WORKERS * G * CH) == 0
    assert n_nodes % NUM_SUBCORES == 0
    groups_per_worker = n_edges // (NUM_WORKERS * G)
    n_chunks = groups_per_worker // CH
    rows_per_sub = n_nodes // NUM_SUBCORES

    mesh = plsc.VectorSubcoreMesh(core_axis_name="c", subcore_axis_name="s")

    out_type = [jax.ShapeDtypeStruct((NUM_CORES, n_nodes, h), jnp.float32)]
    scratch = [
        pltpu.VMEM((CH, G), jnp.int32),
        pltpu.VMEM((CH, G), jnp.int32),
        pltpu.VMEM((G, h), jnp.float32),
        pltpu.SemaphoreType.DMA,
        pltpu.VMEM_SHARED((n_nodes, h), jnp.float32),
    ]
    if with_deg:
        out_type.append(jax.ShapeDtypeStruct((NUM_CORES, n_nodes, h), jnp.float32))
        scratch.append(pltpu.VMEM((G, h), jnp.float32))
        scratch.append(pltpu.VMEM_SHARED((n_nodes, h), jnp.float32))

    def body(y_hbm, src_hbm, dst_hbm, zeros_hbm, ones_hbm, *rest):
        if with_deg:
            (agg_hbm, deg_hbm,
             src_v, dst_v, rows_v, sem, acc_sh, ones_v, deg_sh) = rest
        else:
            (agg_hbm, src_v, dst_v, rows_v, sem, acc_sh) = rest
        cidx = lax.axis_index("c")
        sidx = lax.axis_index("s")
        wid = cidx * NUM_SUBCORES + sidx

        rbase = sidx * rows_per_sub
        pltpu.sync_copy(zeros_hbm.at[pl.ds(rbase, rows_per_sub)],
                        acc_sh.at[pl.ds(rbase, rows_per_sub)])
        if with_deg:
            pltpu.sync_copy(zeros_hbm.at[pl.ds(rbase, rows_per_sub)],
                            deg_sh.at[pl.ds(rbase, rows_per_sub)])
            pltpu.sync_copy(ones_hbm, ones_v)
        plsc.subcore_barrier()

        gbase0 = wid * groups_per_worker

        @pl.loop(0, n_chunks)
        def _(k):
            gb = gbase0 + k * CH
            pltpu.sync_copy(src_hbm.at[pl.ds(gb, CH)], src_v)
            pltpu.sync_copy(dst_hbm.at[pl.ds(gb, CH)], dst_v)
            for j in range(CH):
                pltpu.async_copy(y_hbm.at[src_v.at[j]], rows_v, sem).wait()
                pltpu.sync_copy(rows_v, acc_sh.at[dst_v.at[j]], add=True)
                if with_deg:
                    pltpu.sync_copy(ones_v, deg_sh.at[dst_v.at[j]], add=True)

        plsc.subcore_barrier()
        pltpu.sync_copy(acc_sh.at[pl.ds(rbase, rows_per_sub)],
                        agg_hbm.at[cidx, pl.ds(rbase, rows_per_sub)])
        if with_deg:
            pltpu.sync_copy(deg_sh.at[pl.ds(rbase, rows_per_sub)],
                            deg_hbm.at[cidx, pl.ds(rbase, rows_per_sub)])

    return pl.kernel(body, out_type=tuple(out_type), mesh=mesh,
                     scratch_types=scratch,
                     compiler_params=pltpu.CompilerParams(
                         use_tc_tiling_on_sc=False))


def _proj_body(x_ref, w_ref, o_ref):
    o_ref[...] = jnp.dot(x_ref[...], w_ref[...],
                         preferred_element_type=jnp.float32,
                         precision=lax.Precision.HIGHEST)


def _mid_body(a_ref, d_ref, b_ref, w_ref, o_ref):
    agg = a_ref[0] + a_ref[1]
    deg = jnp.maximum(d_ref[0] + d_ref[1], 1.0)
    hid = jnp.maximum(agg / deg + b_ref[...], 0.0)
    o_ref[...] = jnp.dot(hid, w_ref[...],
                         preferred_element_type=jnp.float32,
                         precision=lax.Precision.HIGHEST)


def _head_body(a_ref, d_ref, b_ref, wp_ref, bp_ref, o_ref):
    agg = a_ref[0] + a_ref[1]
    deg = jnp.maximum(d_ref[0] + d_ref[1], 1.0)
    hid = jnp.maximum(agg / deg + b_ref[...], 0.0)
    pooled = jnp.sum(hid, axis=0, keepdims=True) * (1.0 / hid.shape[0])
    o_ref[...] = jnp.dot(pooled, wp_ref[...],
                         preferred_element_type=jnp.float32,
                         precision=lax.Precision.HIGHEST) + bp_ref[...]


def kernel(x, edge_index, W1, b1, W2, b2, Wp, bp):
    n, d = x.shape
    h = W1.shape[1]
    a = Wp.shape[1]
    e = edge_index.shape[1]

    src2 = edge_index[0].reshape(e // G, G)
    dst2 = edge_index[1].reshape(e // G, G)
    zeros = jnp.zeros((n, h), jnp.float32)
    ones = jnp.ones((G, h), jnp.float32)

    y1 = pl.pallas_call(
        _proj_body,
        out_shape=jax.ShapeDtypeStruct((n, h), jnp.float32),
    )(x, W1)

    agg1, deg = _mp_sc_kernel(n, e, h, True)(y1, src2, dst2, zeros, ones)

    y2 = pl.pallas_call(
        _mid_body,
        out_shape=jax.ShapeDtypeStruct((n, h), jnp.float32),
    )(agg1, deg, b1.reshape(1, h), W2)

    (agg2,) = _mp_sc_kernel(n, e, h, False)(y2, src2, dst2, zeros, ones)

    logits = pl.pallas_call(
        _head_body,
        out_shape=jax.ShapeDtypeStruct((1, a), jnp.float32),
    )(agg2, deg, b2.reshape(1, h), Wp, bp.reshape(1, a))
    return logits.reshape(a)


# speedup vs baseline: 15.6084x; 1.4100x over previous
import functools

import jax
import jax.numpy as jnp
from jax import lax
from jax.experimental import pallas as pl
from jax.experimental.pallas import tpu as pltpu
from jax.experimental.pallas import tpu_sc as plsc

NUM_CORES = 2
NUM_SUBCORES = 16
NUM_WORKERS = NUM_CORES * NUM_SUBCORES

G = 100
CH = 20


def _mp_sc_kernel(n_nodes, n_edges, h, with_deg):
    assert n_edges % (NUM_WORKERS * G * CH) == 0
    assert n_nodes % NUM_SUBCORES == 0
    groups_per_worker = n_edges // (NUM_WORKERS * G)
    n_chunks = groups_per_worker // CH
    rows_per_sub = n_nodes // NUM_SUBCORES

    mesh = plsc.VectorSubcoreMesh(core_axis_name="c", subcore_axis_name="s")

    out_type = [jax.ShapeDtypeStruct((NUM_CORES, n_nodes, h), jnp.float32)]
    scratch = [
        pltpu.VMEM((CH, G), jnp.int32),
        pltpu.VMEM((CH, G), jnp.int32),
        pltpu.VMEM((G, h), jnp.float32),
        pltpu.VMEM((G, h), jnp.float32),
        pltpu.SemaphoreType.DMA,
        pltpu.SemaphoreType.DMA,
        pltpu.VMEM_SHARED((n_nodes, h), jnp.float32),
    ]
    if with_deg:
        out_type.append(jax.ShapeDtypeStruct((NUM_CORES, n_nodes, h), jnp.float32))
        scratch.append(pltpu.VMEM((G, h), jnp.float32))
        scratch.append(pltpu.VMEM_SHARED((n_nodes, h), jnp.float32))

    def body(y_hbm, src_hbm, dst_hbm, zeros_hbm, ones_hbm, *rest):
        if with_deg:
            (agg_hbm, deg_hbm,
             src_v, dst_v, rows0_v, rows1_v, sem0, sem1,
             acc_sh, ones_v, deg_sh) = rest
        else:
            (agg_hbm, src_v, dst_v, rows0_v, rows1_v, sem0, sem1,
             acc_sh) = rest
        rows = (rows0_v, rows1_v)
        sems = (sem0, sem1)
        cidx = lax.axis_index("c")
        sidx = lax.axis_index("s")
        wid = cidx * NUM_SUBCORES + sidx

        rbase = sidx * rows_per_sub
        pltpu.sync_copy(zeros_hbm.at[pl.ds(rbase, rows_per_sub)],
                        acc_sh.at[pl.ds(rbase, rows_per_sub)])
        if with_deg:
            pltpu.sync_copy(zeros_hbm.at[pl.ds(rbase, rows_per_sub)],
                            deg_sh.at[pl.ds(rbase, rows_per_sub)])
            pltpu.sync_copy(ones_hbm, ones_v)
        plsc.subcore_barrier()

        gbase0 = wid * groups_per_worker

        @pl.loop(0, n_chunks)
        def _(k):
            gb = gbase0 + k * CH
            pltpu.sync_copy(src_hbm.at[pl.ds(gb, CH)], src_v)
            pltpu.sync_copy(dst_hbm.at[pl.ds(gb, CH)], dst_v)
            pend = pltpu.async_copy(y_hbm.at[src_v.at[0]], rows[0], sems[0])
            for j in range(CH):
                cur = pend
                if j + 1 < CH:
                    pend = pltpu.async_copy(
                        y_hbm.at[src_v.at[j + 1]],
                        rows[(j + 1) % 2], sems[(j + 1) % 2])
                if with_deg:
                    pltpu.sync_copy(ones_v, deg_sh.at[dst_v.at[j]], add=True)
                cur.wait()
                pltpu.sync_copy(rows[j % 2], acc_sh.at[dst_v.at[j]], add=True)

        plsc.subcore_barrier()
        pltpu.sync_copy(acc_sh.at[pl.ds(rbase, rows_per_sub)],
                        agg_hbm.at[cidx, pl.ds(rbase, rows_per_sub)])
        if with_deg:
            pltpu.sync_copy(deg_sh.at[pl.ds(rbase, rows_per_sub)],
                            deg_hbm.at[cidx, pl.ds(rbase, rows_per_sub)])

    return pl.kernel(body, out_type=tuple(out_type), mesh=mesh,
                     scratch_types=scratch,
                     compiler_params=pltpu.CompilerParams(
                         use_tc_tiling_on_sc=False))


def _proj_body(x_ref, w_ref, o_ref):
    o_ref[...] = jnp.dot(x_ref[...], w_ref[...],
                         preferred_element_type=jnp.float32,
                         precision=lax.Precision.HIGHEST)


def _mid_body(a_ref, d_ref, b_ref, w_ref, o_ref):
    agg = a_ref[0] + a_ref[1]
    deg = jnp.maximum(d_ref[0] + d_ref[1], 1.0)
    hid = jnp.maximum(agg / deg + b_ref[...], 0.0)
    o_ref[...] = jnp.dot(hid, w_ref[...],
                         preferred_element_type=jnp.float32,
                         precision=lax.Precision.HIGHEST)


def _head_body(a_ref, d_ref, b_ref, wp_ref, bp_ref, o_ref):
    agg = a_ref[0] + a_ref[1]
    deg = jnp.maximum(d_ref[0] + d_ref[1], 1.0)
    hid = jnp.maximum(agg / deg + b_ref[...], 0.0)
    pooled = jnp.sum(hid, axis=0, keepdims=True) * (1.0 / hid.shape[0])
    o_ref[...] = jnp.dot(pooled, wp_ref[...],
                         preferred_element_type=jnp.float32,
                         precision=lax.Precision.HIGHEST) + bp_ref[...]


def kernel(x, edge_index, W1, b1, W2, b2, Wp, bp):
    n, d = x.shape
    h = W1.shape[1]
    a = Wp.shape[1]
    e = edge_index.shape[1]

    src2 = edge_index[0].reshape(e // G, G)
    dst2 = edge_index[1].reshape(e // G, G)
    zeros = jnp.zeros((n, h), jnp.float32)
    ones = jnp.ones((G, h), jnp.float32)

    y1 = pl.pallas_call(
        _proj_body,
        out_shape=jax.ShapeDtypeStruct((n, h), jnp.float32),
    )(x, W1)

    agg1, deg = _mp_sc_kernel(n, e, h, True)(y1, src2, dst2, zeros, ones)

    y2 = pl.pallas_call(
        _mid_body,
        out_shape=jax.ShapeDtypeStruct((n, h), jnp.float32),
    )(agg1, deg, b1.reshape(1, h), W2)

    (agg2,) = _mp_sc_kernel(n, e, h, False)(y2, src2, dst2, zeros, ones)

    logits = pl.pallas_call(
        _head_body,
        out_shape=jax.ShapeDtypeStruct((1, a), jnp.float32),
    )(agg2, deg, b2.reshape(1, h), Wp, bp.reshape(1, a))
    return logits.reshape(a)


# speedup vs baseline: 16.1055x; 1.0318x over previous
import functools

import jax
import jax.numpy as jnp
from jax import lax
from jax.experimental import pallas as pl
from jax.experimental.pallas import tpu as pltpu
from jax.experimental.pallas import tpu_sc as plsc

NUM_CORES = 2
NUM_SUBCORES = 16
NUM_WORKERS = NUM_CORES * NUM_SUBCORES

G = 100
CH = 100


def _mp_sc_kernel(n_nodes, n_edges, h, with_deg):
    assert n_edges % (NUM_WORKERS * G * CH) == 0
    assert n_nodes % NUM_SUBCORES == 0
    groups_per_worker = n_edges // (NUM_WORKERS * G)
    n_chunks = groups_per_worker // CH
    rows_per_sub = n_nodes // NUM_SUBCORES

    mesh = plsc.VectorSubcoreMesh(core_axis_name="c", subcore_axis_name="s")

    out_type = [jax.ShapeDtypeStruct((NUM_CORES, n_nodes, h), jnp.float32)]
    scratch = [
        pltpu.VMEM((CH, G), jnp.int32),
        pltpu.VMEM((CH, G), jnp.int32),
        pltpu.VMEM((G, h), jnp.float32),
        pltpu.VMEM((G, h), jnp.float32),
        pltpu.SemaphoreType.DMA,
        pltpu.SemaphoreType.DMA,
        pltpu.VMEM_SHARED((n_nodes, h), jnp.float32),
    ]
    if with_deg:
        out_type.append(jax.ShapeDtypeStruct((NUM_CORES, n_nodes, h), jnp.float32))
        scratch.append(pltpu.VMEM((G, h), jnp.float32))
        scratch.append(pltpu.VMEM_SHARED((n_nodes, h), jnp.float32))

    def body(y_hbm, src_hbm, dst_hbm, zeros_hbm, ones_hbm, *rest):
        if with_deg:
            (agg_hbm, deg_hbm,
             src_v, dst_v, rows0_v, rows1_v, sem0, sem1,
             acc_sh, ones_v, deg_sh) = rest
        else:
            (agg_hbm, src_v, dst_v, rows0_v, rows1_v, sem0, sem1,
             acc_sh) = rest
        rows = (rows0_v, rows1_v)
        sems = (sem0, sem1)
        cidx = lax.axis_index("c")
        sidx = lax.axis_index("s")
        wid = cidx * NUM_SUBCORES + sidx

        rbase = sidx * rows_per_sub
        pltpu.sync_copy(zeros_hbm.at[pl.ds(rbase, rows_per_sub)],
                        acc_sh.at[pl.ds(rbase, rows_per_sub)])
        if with_deg:
            pltpu.sync_copy(zeros_hbm.at[pl.ds(rbase, rows_per_sub)],
                            deg_sh.at[pl.ds(rbase, rows_per_sub)])
            pltpu.sync_copy(ones_hbm, ones_v)
        plsc.subcore_barrier()

        gbase0 = wid * groups_per_worker

        @pl.loop(0, n_chunks)
        def _(k):
            gb = gbase0 + k * CH
            pltpu.sync_copy(src_hbm.at[pl.ds(gb, CH)], src_v)
            pltpu.sync_copy(dst_hbm.at[pl.ds(gb, CH)], dst_v)
            pend = pltpu.async_copy(y_hbm.at[src_v.at[0]], rows[0], sems[0])
            for j in range(CH):
                cur = pend
                if j + 1 < CH:
                    pend = pltpu.async_copy(
                        y_hbm.at[src_v.at[j + 1]],
                        rows[(j + 1) % 2], sems[(j + 1) % 2])
                if with_deg:
                    pltpu.sync_copy(ones_v, deg_sh.at[dst_v.at[j]], add=True)
                cur.wait()
                pltpu.sync_copy(rows[j % 2], acc_sh.at[dst_v.at[j]], add=True)

        plsc.subcore_barrier()
        pltpu.sync_copy(acc_sh.at[pl.ds(rbase, rows_per_sub)],
                        agg_hbm.at[cidx, pl.ds(rbase, rows_per_sub)])
        if with_deg:
            pltpu.sync_copy(deg_sh.at[pl.ds(rbase, rows_per_sub)],
                            deg_hbm.at[cidx, pl.ds(rbase, rows_per_sub)])

    return pl.kernel(body, out_type=tuple(out_type), mesh=mesh,
                     scratch_types=scratch,
                     compiler_params=pltpu.CompilerParams(
                         use_tc_tiling_on_sc=False))


def _proj_body(x_ref, w_ref, o_ref):
    o_ref[...] = jnp.dot(x_ref[...], w_ref[...],
                         preferred_element_type=jnp.float32,
                         precision=lax.Precision.HIGHEST)


def _mid_body(a_ref, d_ref, b_ref, w_ref, o_ref):
    agg = a_ref[0] + a_ref[1]
    deg = jnp.maximum(d_ref[0] + d_ref[1], 1.0)
    hid = jnp.maximum(agg / deg + b_ref[...], 0.0)
    o_ref[...] = jnp.dot(hid, w_ref[...],
                         preferred_element_type=jnp.float32,
                         precision=lax.Precision.HIGHEST)


def _head_body(a_ref, d_ref, b_ref, wp_ref, bp_ref, o_ref):
    agg = a_ref[0] + a_ref[1]
    deg = jnp.maximum(d_ref[0] + d_ref[1], 1.0)
    hid = jnp.maximum(agg / deg + b_ref[...], 0.0)
    pooled = jnp.sum(hid, axis=0, keepdims=True) * (1.0 / hid.shape[0])
    o_ref[...] = jnp.dot(pooled, wp_ref[...],
                         preferred_element_type=jnp.float32,
                         precision=lax.Precision.HIGHEST) + bp_ref[...]


def kernel(x, edge_index, W1, b1, W2, b2, Wp, bp):
    n, d = x.shape
    h = W1.shape[1]
    a = Wp.shape[1]
    e = edge_index.shape[1]

    src2 = edge_index[0].reshape(e // G, G)
    dst2 = edge_index[1].reshape(e // G, G)
    zeros = jnp.zeros((n, h), jnp.float32)
    ones = jnp.ones((G, h), jnp.float32)

    y1 = pl.pallas_call(
        _proj_body,
        out_shape=jax.ShapeDtypeStruct((n, h), jnp.float32),
    )(x, W1)

    agg1, deg = _mp_sc_kernel(n, e, h, True)(y1, src2, dst2, zeros, ones)

    y2 = pl.pallas_call(
        _mid_body,
        out_shape=jax.ShapeDtypeStruct((n, h), jnp.float32),
    )(agg1, deg, b1.reshape(1, h), W2)

    (agg2,) = _mp_sc_kernel(n, e, h, False)(y2, src2, dst2, zeros, ones)

    logits = pl.pallas_call(
        _head_body,
        out_shape=jax.ShapeDtypeStruct((1, a), jnp.float32),
    )(agg2, deg, b2.reshape(1, h), Wp, bp.reshape(1, a))
    return logits.reshape(a)


# speedup vs baseline: 19.3698x; 1.2027x over previous
import functools

import jax
import jax.numpy as jnp
from jax import lax
from jax.experimental import pallas as pl
from jax.experimental.pallas import tpu as pltpu
from jax.experimental.pallas import tpu_sc as plsc

NUM_CORES = 2
NUM_SUBCORES = 16
NUM_WORKERS = NUM_CORES * NUM_SUBCORES

G = 100
CH = 100


def _mp_sc_kernel(n_nodes, n_edges, h, with_deg):
    assert n_edges % (NUM_WORKERS * G * CH) == 0
    assert n_nodes % NUM_SUBCORES == 0
    groups_per_worker = n_edges // (NUM_WORKERS * G)
    n_chunks = groups_per_worker // CH
    rows_per_sub = n_nodes // NUM_SUBCORES

    mesh = plsc.VectorSubcoreMesh(core_axis_name="c", subcore_axis_name="s")

    nbuf = 4
    out_type = [jax.ShapeDtypeStruct((NUM_CORES, n_nodes, h), jnp.float32)]
    scratch = [
        pltpu.VMEM((CH, G), jnp.int32),
        pltpu.VMEM((CH, G), jnp.int32),
        [pltpu.VMEM((G, h), jnp.float32) for _ in range(nbuf)],
        [pltpu.SemaphoreType.DMA for _ in range(nbuf)],
        [pltpu.SemaphoreType.DMA for _ in range(nbuf)],
        pltpu.SemaphoreType.DMA,
        pltpu.VMEM_SHARED((n_nodes, h), jnp.float32),
    ]
    if with_deg:
        out_type.append(jax.ShapeDtypeStruct((NUM_CORES, n_nodes, h), jnp.float32))
        scratch.append(pltpu.VMEM((G, h), jnp.float32))
        scratch.append(pltpu.VMEM_SHARED((n_nodes, h), jnp.float32))

    def body(y_hbm, src_hbm, dst_hbm, zeros_hbm, ones_hbm, *rest):
        if with_deg:
            (agg_hbm, deg_hbm,
             src_v, dst_v, rows, gsems, ssems, dsem,
             acc_sh, ones_v, deg_sh) = rest
        else:
            (agg_hbm, src_v, dst_v, rows, gsems, ssems, dsem,
             acc_sh) = rest
        cidx = lax.axis_index("c")
        sidx = lax.axis_index("s")
        wid = cidx * NUM_SUBCORES + sidx

        rbase = sidx * rows_per_sub
        pltpu.sync_copy(zeros_hbm.at[pl.ds(rbase, rows_per_sub)],
                        acc_sh.at[pl.ds(rbase, rows_per_sub)])
        if with_deg:
            pltpu.sync_copy(zeros_hbm.at[pl.ds(rbase, rows_per_sub)],
                            deg_sh.at[pl.ds(rbase, rows_per_sub)])
            pltpu.sync_copy(ones_hbm, ones_v)
        plsc.subcore_barrier()

        gbase0 = wid * groups_per_worker

        deg_win = 8

        @pl.loop(0, n_chunks)
        def _(k):
            gb = gbase0 + k * CH
            pltpu.sync_copy(src_hbm.at[pl.ds(gb, CH)], src_v)
            pltpu.sync_copy(dst_hbm.at[pl.ds(gb, CH)], dst_v)
            gath = [None] * CH
            scat = [None] * CH
            degs = [None] * CH
            lead = nbuf - 1
            for j in range(min(lead, CH)):
                gath[j] = pltpu.async_copy(
                    y_hbm.at[src_v.at[j]], rows[j % nbuf], gsems[j % nbuf])
            for j in range(CH):
                if j > 0:
                    scat[j - 1].wait()
                if j + lead < CH:
                    b = (j + lead) % nbuf
                    gath[j + lead] = pltpu.async_copy(
                        y_hbm.at[src_v.at[j + lead]], rows[b], gsems[b])
                gath[j].wait()
                scat[j] = pltpu.async_copy(
                    rows[j % nbuf], acc_sh.at[dst_v.at[j]],
                    ssems[j % nbuf], add=True)
                if with_deg:
                    degs[j] = pltpu.async_copy(
                        ones_v, deg_sh.at[dst_v.at[j]], dsem, add=True)
                    if j >= deg_win:
                        degs[j - deg_win].wait()
            scat[CH - 1].wait()
            if with_deg:
                for j in range(max(CH - deg_win, 0), CH):
                    degs[j].wait()

        plsc.subcore_barrier()
        pltpu.sync_copy(acc_sh.at[pl.ds(rbase, rows_per_sub)],
                        agg_hbm.at[cidx, pl.ds(rbase, rows_per_sub)])
        if with_deg:
            pltpu.sync_copy(deg_sh.at[pl.ds(rbase, rows_per_sub)],
                            deg_hbm.at[cidx, pl.ds(rbase, rows_per_sub)])

    return pl.kernel(body, out_type=tuple(out_type), mesh=mesh,
                     scratch_types=scratch,
                     compiler_params=pltpu.CompilerParams(
                         use_tc_tiling_on_sc=False))


def _proj_body(x_ref, w_ref, o_ref):
    o_ref[...] = jnp.dot(x_ref[...], w_ref[...],
                         preferred_element_type=jnp.float32,
                         precision=lax.Precision.HIGHEST)


def _mid_body(a_ref, d_ref, b_ref, w_ref, o_ref):
    agg = a_ref[0] + a_ref[1]
    deg = jnp.maximum(d_ref[0] + d_ref[1], 1.0)
    hid = jnp.maximum(agg / deg + b_ref[...], 0.0)
    o_ref[...] = jnp.dot(hid, w_ref[...],
                         preferred_element_type=jnp.float32,
                         precision=lax.Precision.HIGHEST)


def _head_body(a_ref, d_ref, b_ref, wp_ref, bp_ref, o_ref):
    agg = a_ref[0] + a_ref[1]
    deg = jnp.maximum(d_ref[0] + d_ref[1], 1.0)
    hid = jnp.maximum(agg / deg + b_ref[...], 0.0)
    pooled = jnp.sum(hid, axis=0, keepdims=True) * (1.0 / hid.shape[0])
    o_ref[...] = jnp.dot(pooled, wp_ref[...],
                         preferred_element_type=jnp.float32,
                         precision=lax.Precision.HIGHEST) + bp_ref[...]


def kernel(x, edge_index, W1, b1, W2, b2, Wp, bp):
    n, d = x.shape
    h = W1.shape[1]
    a = Wp.shape[1]
    e = edge_index.shape[1]

    src2 = edge_index[0].reshape(e // G, G)
    dst2 = edge_index[1].reshape(e // G, G)
    zeros = jnp.zeros((n, h), jnp.float32)
    ones = jnp.ones((G, h), jnp.float32)

    y1 = pl.pallas_call(
        _proj_body,
        out_shape=jax.ShapeDtypeStruct((n, h), jnp.float32),
    )(x, W1)

    agg1, deg = _mp_sc_kernel(n, e, h, True)(y1, src2, dst2, zeros, ones)

    y2 = pl.pallas_call(
        _mid_body,
        out_shape=jax.ShapeDtypeStruct((n, h), jnp.float32),
    )(agg1, deg, b1.reshape(1, h), W2)

    (agg2,) = _mp_sc_kernel(n, e, h, False)(y2, src2, dst2, zeros, ones)

    logits = pl.pallas_call(
        _head_body,
        out_shape=jax.ShapeDtypeStruct((1, a), jnp.float32),
    )(agg2, deg, b2.reshape(1, h), Wp, bp.reshape(1, a))
    return logits.reshape(a)


# speedup vs baseline: 19.3788x; 1.0005x over previous
import functools

import jax
import jax.numpy as jnp
from jax import lax
from jax.experimental import pallas as pl
from jax.experimental.pallas import tpu as pltpu
from jax.experimental.pallas import tpu_sc as plsc

NUM_CORES = 2
NUM_SUBCORES = 16
NUM_WORKERS = NUM_CORES * NUM_SUBCORES

G = 100
CH = 100


def _mp_sc_kernel(n_nodes, n_edges, h, with_deg):
    assert n_edges % (NUM_WORKERS * G * CH) == 0
    assert n_nodes % NUM_SUBCORES == 0
    groups_per_worker = n_edges // (NUM_WORKERS * G)
    n_chunks = groups_per_worker // CH
    rows_per_sub = n_nodes // NUM_SUBCORES

    mesh = plsc.VectorSubcoreMesh(core_axis_name="c", subcore_axis_name="s")

    nbuf = 4
    out_type = [jax.ShapeDtypeStruct((NUM_CORES, n_nodes, h), jnp.float32)]
    scratch = [
        pltpu.VMEM((CH, G), jnp.int32),
        pltpu.VMEM((CH, G), jnp.int32),
        [pltpu.VMEM((G, h), jnp.float32) for _ in range(nbuf)],
        [pltpu.SemaphoreType.DMA for _ in range(nbuf)],
        [pltpu.SemaphoreType.DMA for _ in range(nbuf)],
        pltpu.SemaphoreType.DMA,
        pltpu.VMEM_SHARED((n_nodes, h), jnp.float32),
    ]
    if with_deg:
        out_type.append(jax.ShapeDtypeStruct((NUM_CORES, n_nodes, h), jnp.float32))
        scratch.append(pltpu.VMEM((G, h), jnp.float32))
        scratch.append(pltpu.VMEM_SHARED((n_nodes, h), jnp.float32))

    def body(y_hbm, src_hbm, dst_hbm, zeros_hbm, ones_hbm, *rest):
        if with_deg:
            (agg_hbm, deg_hbm,
             src_v, dst_v, rows, gsems, ssems, dsem,
             acc_sh, ones_v, deg_sh) = rest
        else:
            (agg_hbm, src_v, dst_v, rows, gsems, ssems, dsem,
             acc_sh) = rest
        cidx = lax.axis_index("c")
        sidx = lax.axis_index("s")
        wid = cidx * NUM_SUBCORES + sidx

        rbase = sidx * rows_per_sub
        pltpu.sync_copy(zeros_hbm.at[pl.ds(rbase, rows_per_sub)],
                        acc_sh.at[pl.ds(rbase, rows_per_sub)])
        if with_deg:
            pltpu.sync_copy(zeros_hbm.at[pl.ds(rbase, rows_per_sub)],
                            deg_sh.at[pl.ds(rbase, rows_per_sub)])
            pltpu.sync_copy(ones_hbm, ones_v)
        plsc.subcore_barrier()

        gbase0 = wid * groups_per_worker

        deg_win = 8

        @pl.loop(0, n_chunks)
        def _(k):
            gb = gbase0 + k * CH
            pltpu.sync_copy(src_hbm.at[pl.ds(gb, CH)], src_v)
            pltpu.sync_copy(dst_hbm.at[pl.ds(gb, CH)], dst_v)
            lead = nbuf - 1
            gath = [None] * CH
            for j in range(min(lead, CH)):
                gath[j] = pltpu.async_copy(
                    y_hbm.at[src_v.at[j]], rows[j % nbuf], gsems[j % nbuf])
            for j in range(CH):
                if j + lead < CH:
                    b = (j + lead) % nbuf
                    gath[j + lead] = pltpu.async_copy(
                        y_hbm.at[src_v.at[j + lead]], rows[b], gsems[b])
                if with_deg:
                    pltpu.sync_copy(ones_v, deg_sh.at[dst_v.at[j]], add=True)
                gath[j].wait()
                pltpu.sync_copy(rows[j % nbuf], acc_sh.at[dst_v.at[j]], add=True)

        plsc.subcore_barrier()
        pltpu.sync_copy(acc_sh.at[pl.ds(rbase, rows_per_sub)],
                        agg_hbm.at[cidx, pl.ds(rbase, rows_per_sub)])
        if with_deg:
            pltpu.sync_copy(deg_sh.at[pl.ds(rbase, rows_per_sub)],
                            deg_hbm.at[cidx, pl.ds(rbase, rows_per_sub)])

    return pl.kernel(body, out_type=tuple(out_type), mesh=mesh,
                     scratch_types=scratch,
                     compiler_params=pltpu.CompilerParams(
                         use_tc_tiling_on_sc=False))


def _proj_body(x_ref, w_ref, o_ref):
    o_ref[...] = jnp.dot(x_ref[...], w_ref[...],
                         preferred_element_type=jnp.float32,
                         precision=lax.Precision.HIGHEST)


def _mid_body(a_ref, d_ref, b_ref, w_ref, o_ref):
    agg = a_ref[0] + a_ref[1]
    deg = jnp.maximum(d_ref[0] + d_ref[1], 1.0)
    hid = jnp.maximum(agg / deg + b_ref[...], 0.0)
    o_ref[...] = jnp.dot(hid, w_ref[...],
                         preferred_element_type=jnp.float32,
                         precision=lax.Precision.HIGHEST)


def _head_body(a_ref, d_ref, b_ref, wp_ref, bp_ref, o_ref):
    agg = a_ref[0] + a_ref[1]
    deg = jnp.maximum(d_ref[0] + d_ref[1], 1.0)
    hid = jnp.maximum(agg / deg + b_ref[...], 0.0)
    pooled = jnp.sum(hid, axis=0, keepdims=True) * (1.0 / hid.shape[0])
    o_ref[...] = jnp.dot(pooled, wp_ref[...],
                         preferred_element_type=jnp.float32,
                         precision=lax.Precision.HIGHEST) + bp_ref[...]


def kernel(x, edge_index, W1, b1, W2, b2, Wp, bp):
    n, d = x.shape
    h = W1.shape[1]
    a = Wp.shape[1]
    e = edge_index.shape[1]

    src2 = edge_index[0].reshape(e // G, G)
    dst2 = edge_index[1].reshape(e // G, G)
    zeros = jnp.zeros((n, h), jnp.float32)
    ones = jnp.ones((G, h), jnp.float32)

    y1 = pl.pallas_call(
        _proj_body,
        out_shape=jax.ShapeDtypeStruct((n, h), jnp.float32),
    )(x, W1)

    agg1, deg = _mp_sc_kernel(n, e, h, True)(y1, src2, dst2, zeros, ones)

    y2 = pl.pallas_call(
        _mid_body,
        out_shape=jax.ShapeDtypeStruct((n, h), jnp.float32),
    )(agg1, deg, b1.reshape(1, h), W2)

    (agg2,) = _mp_sc_kernel(n, e, h, False)(y2, src2, dst2, zeros, ones)

    logits = pl.pallas_call(
        _head_body,
        out_shape=jax.ShapeDtypeStruct((1, a), jnp.float32),
    )(agg2, deg, b2.reshape(1, h), Wp, bp.reshape(1, a))
    return logits.reshape(a)


# speedup vs baseline: 21.7694x; 1.1234x over previous
import functools

import jax
import jax.numpy as jnp
from jax import lax
from jax.experimental import pallas as pl
from jax.experimental.pallas import tpu as pltpu
from jax.experimental.pallas import tpu_sc as plsc

NUM_CORES = 2
NUM_SUBCORES = 16
NUM_WORKERS = NUM_CORES * NUM_SUBCORES

G = 125
CH = 80


def _mp_sc_kernel(n_nodes, n_edges, h, with_deg):
    assert n_edges % (NUM_WORKERS * G * CH) == 0
    assert n_nodes % NUM_SUBCORES == 0
    groups_per_worker = n_edges // (NUM_WORKERS * G)
    n_chunks = groups_per_worker // CH
    assert n_chunks == 1
    rows_per_sub = n_nodes // NUM_SUBCORES

    mesh = plsc.VectorSubcoreMesh(core_axis_name="c", subcore_axis_name="s")

    nbuf = 6
    out_type = [jax.ShapeDtypeStruct((NUM_CORES, n_nodes, h), jnp.float32)]
    scratch = [
        pltpu.VMEM((CH, G), jnp.int32),
        pltpu.VMEM((CH, G), jnp.int32),
        [pltpu.VMEM((G, h), jnp.float32) for _ in range(nbuf)],
        [pltpu.SemaphoreType.DMA for _ in range(nbuf)],
        [pltpu.SemaphoreType.DMA for _ in range(nbuf)],
        pltpu.SemaphoreType.DMA,
        pltpu.VMEM_SHARED((n_nodes, h), jnp.float32),
    ]
    if with_deg:
        out_type.append(jax.ShapeDtypeStruct((NUM_CORES, n_nodes, h), jnp.float32))
        scratch.append(pltpu.VMEM((G, h), jnp.float32))
        scratch.append(pltpu.VMEM_SHARED((n_nodes, h), jnp.float32))

    def body(y_hbm, src_hbm, dst_hbm, zeros_hbm, ones_hbm, *rest):
        if with_deg:
            (agg_hbm, deg_hbm,
             src_v, dst_v, rows, gsems, ssems, dsem,
             acc_sh, ones_v, deg_sh) = rest
        else:
            (agg_hbm, src_v, dst_v, rows, gsems, ssems, dsem,
             acc_sh) = rest
        cidx = lax.axis_index("c")
        sidx = lax.axis_index("s")
        wid = cidx * NUM_SUBCORES + sidx

        rbase = sidx * rows_per_sub
        zcopies = [pltpu.async_copy(zeros_hbm.at[pl.ds(rbase, rows_per_sub)],
                                    acc_sh.at[pl.ds(rbase, rows_per_sub)],
                                    dsem)]
        if with_deg:
            zcopies.append(pltpu.async_copy(
                zeros_hbm.at[pl.ds(rbase, rows_per_sub)],
                deg_sh.at[pl.ds(rbase, rows_per_sub)], dsem))
            zcopies.append(pltpu.async_copy(ones_hbm, ones_v, dsem))

        gbase0 = wid * groups_per_worker

        deg_win = 8

        @pl.loop(0, n_chunks)
        def _(k):
            gb = gbase0 + k * CH
            pltpu.sync_copy(src_hbm.at[pl.ds(gb, CH)], src_v)
            pltpu.sync_copy(dst_hbm.at[pl.ds(gb, CH)], dst_v)
            for zc in zcopies:
                zc.wait()
            plsc.subcore_barrier()
            lead = nbuf - 1
            gath = [None] * CH
            for j in range(min(lead, CH)):
                gath[j] = pltpu.async_copy(
                    y_hbm.at[src_v.at[j]], rows[j % nbuf], gsems[j % nbuf])
            for j in range(CH):
                if j + lead < CH:
                    b = (j + lead) % nbuf
                    gath[j + lead] = pltpu.async_copy(
                        y_hbm.at[src_v.at[j + lead]], rows[b], gsems[b])
                if with_deg:
                    pltpu.sync_copy(ones_v, deg_sh.at[dst_v.at[j]], add=True)
                gath[j].wait()
                pltpu.sync_copy(rows[j % nbuf], acc_sh.at[dst_v.at[j]], add=True)

        plsc.subcore_barrier()
        wb = [pltpu.async_copy(acc_sh.at[pl.ds(rbase, rows_per_sub)],
                               agg_hbm.at[cidx, pl.ds(rbase, rows_per_sub)],
                               dsem)]
        if with_deg:
            wb.append(pltpu.async_copy(
                deg_sh.at[pl.ds(rbase, rows_per_sub)],
                deg_hbm.at[cidx, pl.ds(rbase, rows_per_sub)], dsem))
        for w in wb:
            w.wait()

    return pl.kernel(body, out_type=tuple(out_type), mesh=mesh,
                     scratch_types=scratch,
                     compiler_params=pltpu.CompilerParams(
                         use_tc_tiling_on_sc=False))


def _proj_body(x_ref, w_ref, o_ref):
    o_ref[...] = jnp.dot(x_ref[...], w_ref[...],
                         preferred_element_type=jnp.float32,
                         precision=lax.Precision.HIGHEST)


def _mid_body(a_ref, d_ref, b_ref, w_ref, o_ref):
    agg = a_ref[0] + a_ref[1]
    deg = jnp.maximum(d_ref[0] + d_ref[1], 1.0)
    hid = jnp.maximum(agg / deg + b_ref[...], 0.0)
    o_ref[...] = jnp.dot(hid, w_ref[...],
                         preferred_element_type=jnp.float32,
                         precision=lax.Precision.HIGHEST)


def _head_body(a_ref, d_ref, b_ref, wp_ref, bp_ref, o_ref):
    agg = a_ref[0] + a_ref[1]
    deg = jnp.maximum(d_ref[0] + d_ref[1], 1.0)
    hid = jnp.maximum(agg / deg + b_ref[...], 0.0)
    pooled = jnp.sum(hid, axis=0, keepdims=True) * (1.0 / hid.shape[0])
    o_ref[...] = jnp.dot(pooled, wp_ref[...],
                         preferred_element_type=jnp.float32,
                         precision=lax.Precision.HIGHEST) + bp_ref[...]


def kernel(x, edge_index, W1, b1, W2, b2, Wp, bp):
    n, d = x.shape
    h = W1.shape[1]
    a = Wp.shape[1]
    e = edge_index.shape[1]

    src2 = edge_index[0].reshape(e // G, G)
    dst2 = edge_index[1].reshape(e // G, G)
    zeros = jnp.zeros((n, h), jnp.float32)
    ones = jnp.ones((G, h), jnp.float32)

    y1 = pl.pallas_call(
        _proj_body,
        out_shape=jax.ShapeDtypeStruct((n, h), jnp.float32),
    )(x, W1)

    agg1, deg = _mp_sc_kernel(n, e, h, True)(y1, src2, dst2, zeros, ones)

    y2 = pl.pallas_call(
        _mid_body,
        out_shape=jax.ShapeDtypeStruct((n, h), jnp.float32),
    )(agg1, deg, b1.reshape(1, h), W2)

    (agg2,) = _mp_sc_kernel(n, e, h, False)(y2, src2, dst2, zeros, ones)

    logits = pl.pallas_call(
        _head_body,
        out_shape=jax.ShapeDtypeStruct((1, a), jnp.float32),
    )(agg2, deg, b2.reshape(1, h), Wp, bp.reshape(1, a))
    return logits.reshape(a)


# speedup vs baseline: 23.2847x; 1.0696x over previous
import jax
import jax.numpy as jnp
from jax import lax
from jax.experimental import pallas as pl
from jax.experimental.pallas import tpu as pltpu
from jax.experimental.pallas import tpu_sc as plsc

NUM_CORES = 2
NUM_SUBCORES = 16
NUM_WORKERS = NUM_CORES * NUM_SUBCORES

G = 125
CH = 80
NBUF = 6


def _seg_sum_kernel(n_nodes, n_edges, width):
    assert n_edges % (NUM_WORKERS * G * CH) == 0
    assert n_edges // (NUM_WORKERS * G) == CH
    assert n_nodes % NUM_SUBCORES == 0
    rows_per_sub = n_nodes // NUM_SUBCORES

    mesh = plsc.VectorSubcoreMesh(core_axis_name="c", subcore_axis_name="s")

    scratch = [
        pltpu.VMEM((CH, G), jnp.int32),
        pltpu.VMEM((CH, G), jnp.int32),
        [pltpu.VMEM((G, width), jnp.float32) for _ in range(NBUF)],
        [pltpu.SemaphoreType.DMA for _ in range(NBUF)],
        pltpu.SemaphoreType.DMA,
        pltpu.VMEM_SHARED((n_nodes, width), jnp.float32),
    ]

    def body(y_hbm, src_hbm, dst_hbm, zeros_hbm,
             parts_hbm, src_v, dst_v, rows, gsems, hsem, acc_sh):
        cidx = lax.axis_index("c")
        sidx = lax.axis_index("s")
        wid = cidx * NUM_SUBCORES + sidx

        rbase = sidx * rows_per_sub
        zcopy = pltpu.async_copy(zeros_hbm.at[pl.ds(rbase, rows_per_sub)],
                                 acc_sh.at[pl.ds(rbase, rows_per_sub)], hsem)

        gb = wid * CH
        pltpu.sync_copy(src_hbm.at[pl.ds(gb, CH)], src_v)
        pltpu.sync_copy(dst_hbm.at[pl.ds(gb, CH)], dst_v)
        zcopy.wait()
        plsc.subcore_barrier()

        lead = NBUF - 1
        gath = [None] * CH
        for j in range(min(lead, CH)):
            gath[j] = pltpu.async_copy(
                y_hbm.at[src_v.at[j]], rows[j % NBUF], gsems[j % NBUF])
        for j in range(CH):
            if j + lead < CH:
                b = (j + lead) % NBUF
                gath[j + lead] = pltpu.async_copy(
                    y_hbm.at[src_v.at[j + lead]], rows[b], gsems[b])
            gath[j].wait()
            pltpu.sync_copy(rows[j % NBUF], acc_sh.at[dst_v.at[j]], add=True)

        plsc.subcore_barrier()
        pltpu.async_copy(acc_sh.at[pl.ds(rbase, rows_per_sub)],
                         parts_hbm.at[cidx, pl.ds(rbase, rows_per_sub)],
                         hsem).wait()

    return pl.kernel(
        body,
        out_type=jax.ShapeDtypeStruct((NUM_CORES, n_nodes, width),
                                      jnp.float32),
        mesh=mesh, scratch_types=scratch,
        compiler_params=pltpu.CompilerParams(use_tc_tiling_on_sc=False))


def _proj_body(x_ref, w_ref, o_ref):
    n = x_ref.shape[0]
    h = w_ref.shape[1]
    y = jnp.dot(x_ref[...], w_ref[...],
                preferred_element_type=jnp.float32,
                precision=lax.Precision.HIGHEST)
    o_ref[...] = jnp.concatenate(
        [y, jnp.ones((n, 1), jnp.float32),
         jnp.zeros((n, h - 1), jnp.float32)], axis=1)


def _mid_body(p_ref, b_ref, w_ref, o_ref):
    h = w_ref.shape[0]
    agg = p_ref[0, :, :h] + p_ref[1, :, :h]
    deg = jnp.maximum(p_ref[0, :, h:h + 1] + p_ref[1, :, h:h + 1], 1.0)
    hid = jnp.maximum(agg / deg + b_ref[...], 0.0)
    o_ref[...] = jnp.dot(hid, w_ref[...],
                         preferred_element_type=jnp.float32,
                         precision=lax.Precision.HIGHEST)


def _head_body(a_ref, p_ref, b_ref, wp_ref, bp_ref, o_ref):
    h = a_ref.shape[2]
    agg = a_ref[0] + a_ref[1]
    deg = jnp.maximum(p_ref[0, :, h:h + 1] + p_ref[1, :, h:h + 1], 1.0)
    hid = jnp.maximum(agg / deg + b_ref[...], 0.0)
    pooled = jnp.sum(hid, axis=0, keepdims=True) * (1.0 / hid.shape[0])
    o_ref[...] = jnp.dot(pooled, wp_ref[...],
                         preferred_element_type=jnp.float32,
                         precision=lax.Precision.HIGHEST) + bp_ref[...]


def kernel(x, edge_index, W1, b1, W2, b2, Wp, bp):
    n, d = x.shape
    h = W1.shape[1]
    a = Wp.shape[1]
    e = edge_index.shape[1]
    w1x = 2 * h

    src2 = edge_index[0].reshape(e // G, G)
    dst2 = edge_index[1].reshape(e // G, G)
    zeros_w = jnp.zeros((n, w1x), jnp.float32)
    zeros_h = jnp.zeros((n, h), jnp.float32)

    y1 = pl.pallas_call(
        _proj_body,
        out_shape=jax.ShapeDtypeStruct((n, w1x), jnp.float32),
    )(x, W1)

    parts1 = _seg_sum_kernel(n, e, w1x)(y1, src2, dst2, zeros_w)

    y2 = pl.pallas_call(
        _mid_body,
        out_shape=jax.ShapeDtypeStruct((n, h), jnp.float32),
    )(parts1, b1.reshape(1, h), W2)

    parts2 = _seg_sum_kernel(n, e, h)(y2, src2, dst2, zeros_h)

    logits = pl.pallas_call(
        _head_body,
        out_shape=jax.ShapeDtypeStruct((1, a), jnp.float32),
    )(parts2, parts1, b2.reshape(1, h), Wp, bp.reshape(1, a))
    return logits.reshape(a)


# speedup vs baseline: 23.3471x; 1.0027x over previous
import jax
import jax.numpy as jnp
from jax import lax
from jax.experimental import pallas as pl
from jax.experimental.pallas import tpu as pltpu
from jax.experimental.pallas import tpu_sc as plsc

NUM_CORES = 2
NUM_SUBCORES = 16
NUM_WORKERS = NUM_CORES * NUM_SUBCORES

G = 125
CH = 80
NBUF = 12


def _seg_sum_kernel(n_nodes, n_edges, width):
    assert n_edges % (NUM_WORKERS * G * CH) == 0
    assert n_edges // (NUM_WORKERS * G) == CH
    assert n_nodes % NUM_SUBCORES == 0
    rows_per_sub = n_nodes // NUM_SUBCORES

    mesh = plsc.VectorSubcoreMesh(core_axis_name="c", subcore_axis_name="s")

    scratch = [
        pltpu.VMEM((CH, G), jnp.int32),
        pltpu.VMEM((CH, G), jnp.int32),
        [pltpu.VMEM((G, width), jnp.float32) for _ in range(NBUF)],
        [pltpu.SemaphoreType.DMA for _ in range(NBUF)],
        pltpu.SemaphoreType.DMA,
        pltpu.VMEM_SHARED((n_nodes, width), jnp.float32),
    ]

    def body(y_hbm, src_hbm, dst_hbm, zeros_hbm,
             parts_hbm, src_v, dst_v, rows, gsems, hsem, acc_sh):
        cidx = lax.axis_index("c")
        sidx = lax.axis_index("s")
        wid = cidx * NUM_SUBCORES + sidx

        rbase = sidx * rows_per_sub
        zcopy = pltpu.async_copy(zeros_hbm.at[pl.ds(rbase, rows_per_sub)],
                                 acc_sh.at[pl.ds(rbase, rows_per_sub)], hsem)

        gb = wid * CH
        pltpu.sync_copy(src_hbm.at[pl.ds(gb, CH)], src_v)
        pltpu.sync_copy(dst_hbm.at[pl.ds(gb, CH)], dst_v)
        zcopy.wait()
        plsc.subcore_barrier()

        lead = NBUF - 1
        gath = [None] * CH
        for j in range(min(lead, CH)):
            gath[j] = pltpu.async_copy(
                y_hbm.at[src_v.at[j]], rows[j % NBUF], gsems[j % NBUF])
        for j in range(CH):
            if j + lead < CH:
                b = (j + lead) % NBUF
                gath[j + lead] = pltpu.async_copy(
                    y_hbm.at[src_v.at[j + lead]], rows[b], gsems[b])
            gath[j].wait()
            pltpu.sync_copy(rows[j % NBUF], acc_sh.at[dst_v.at[j]], add=True)

        plsc.subcore_barrier()
        pltpu.async_copy(acc_sh.at[pl.ds(rbase, rows_per_sub)],
                         parts_hbm.at[cidx, pl.ds(rbase, rows_per_sub)],
                         hsem).wait()

    return pl.kernel(
        body,
        out_type=jax.ShapeDtypeStruct((NUM_CORES, n_nodes, width),
                                      jnp.float32),
        mesh=mesh, scratch_types=scratch,
        compiler_params=pltpu.CompilerParams(use_tc_tiling_on_sc=False))


def _proj_body(x_ref, w_ref, o_ref):
    n = x_ref.shape[0]
    h = w_ref.shape[1]
    y = jnp.dot(x_ref[...], w_ref[...],
                preferred_element_type=jnp.float32,
                precision=lax.Precision.HIGHEST)
    o_ref[...] = jnp.concatenate(
        [y, jnp.ones((n, 1), jnp.float32),
         jnp.zeros((n, h - 1), jnp.float32)], axis=1)


def _mid_body(p_ref, b_ref, w_ref, o_ref):
    h = w_ref.shape[0]
    agg = p_ref[0, :, :h] + p_ref[1, :, :h]
    deg = jnp.maximum(p_ref[0, :, h:h + 1] + p_ref[1, :, h:h + 1], 1.0)
    hid = jnp.maximum(agg / deg + b_ref[...], 0.0)
    o_ref[...] = jnp.dot(hid, w_ref[...],
                         preferred_element_type=jnp.float32,
                         precision=lax.Precision.HIGHEST)


def _head_body(a_ref, p_ref, b_ref, wp_ref, bp_ref, o_ref):
    h = a_ref.shape[2]
    agg = a_ref[0] + a_ref[1]
    deg = jnp.maximum(p_ref[0, :, h:h + 1] + p_ref[1, :, h:h + 1], 1.0)
    hid = jnp.maximum(agg / deg + b_ref[...], 0.0)
    pooled = jnp.sum(hid, axis=0, keepdims=True) * (1.0 / hid.shape[0])
    o_ref[...] = jnp.dot(pooled, wp_ref[...],
                         preferred_element_type=jnp.float32,
                         precision=lax.Precision.HIGHEST) + bp_ref[...]


def kernel(x, edge_index, W1, b1, W2, b2, Wp, bp):
    n, d = x.shape
    h = W1.shape[1]
    a = Wp.shape[1]
    e = edge_index.shape[1]
    w1x = 2 * h

    src2 = edge_index[0].reshape(e // G, G)
    dst2 = edge_index[1].reshape(e // G, G)
    zeros_w = jnp.zeros((n, w1x), jnp.float32)
    zeros_h = jnp.zeros((n, h), jnp.float32)

    y1 = pl.pallas_call(
        _proj_body,
        out_shape=jax.ShapeDtypeStruct((n, w1x), jnp.float32),
    )(x, W1)

    parts1 = _seg_sum_kernel(n, e, w1x)(y1, src2, dst2, zeros_w)

    y2 = pl.pallas_call(
        _mid_body,
        out_shape=jax.ShapeDtypeStruct((n, h), jnp.float32),
    )(parts1, b1.reshape(1, h), W2)

    parts2 = _seg_sum_kernel(n, e, h)(y2, src2, dst2, zeros_h)

    logits = pl.pallas_call(
        _head_body,
        out_shape=jax.ShapeDtypeStruct((1, a), jnp.float32),
    )(parts2, parts1, b2.reshape(1, h), Wp, bp.reshape(1, a))
    return logits.reshape(a)
